# baseline (speedup 1.0000x reference)
import jax
import jax.numpy as jnp
from jax import lax
from jax.experimental import pallas as pl
from jax.experimental.pallas import tpu as pltpu
from jax.experimental.pallas import tpu_sc as plsc

NN = 10000
NPAD = 10240
EDGES = 320000
ETOT = EDGES + NN
NCORE = 2
NSUB = 16
NTILE = NCORE * NSUB
CHUNK = 128
CPT = 82
EPT = CPT * CHUNK
EPAD = EPT * NTILE
ROWS_PER_TILE = NPAD // NSUB

U1W = 80
U2W = 48
HBW = 64
BLK = 1024


def _tc_pre_body(x_ref, w1_ref, w1s_ref, acat_ref, hb_ref, asd_ref):
    h = jnp.dot(x_ref[...], w1_ref[...], preferred_element_type=jnp.float32)
    asd_ref[...] = jnp.dot(h, acat_ref[...], preferred_element_type=jnp.float32)
    hb_ref[...] = jnp.dot(x_ref[...], w1s_ref[...],
                          preferred_element_type=jnp.float32).astype(jnp.bfloat16)


def _tc_mid_body(u_ref, b1_ref, w2p_ref, w2s_ref, a2_ref, e16_ref,
                 hb_ref, asd2_ref):
    u = u_ref[0] + u_ref[1]
    den = jnp.dot(u[:, 64:80], e16_ref[...], preferred_element_type=jnp.float32)
    h1 = jnp.maximum(u[:, :64] / (den + 1e-16) + b1_ref[...], 0.0)
    h2 = jnp.dot(h1, w2p_ref[...], preferred_element_type=jnp.float32)
    asd2_ref[...] = jnp.dot(h2, a2_ref[...], preferred_element_type=jnp.float32)
    hb_ref[...] = jnp.dot(h1, w2s_ref[...],
                          preferred_element_type=jnp.float32).astype(jnp.bfloat16)


def _tc_post_body(u_ref, b2_ref, e1_ref, out_ref):
    u = u_ref[0] + u_ref[1]
    den = jnp.dot(u[:, 40:48], e1_ref[...], preferred_element_type=jnp.float32)
    logits = u[:, :40] / (den + 1e-16) + b2_ref[...]
    m = jnp.max(logits, axis=1, keepdims=True)
    p = logits - m
    out_ref[...] = p - jnp.log(jnp.sum(jnp.exp(p), axis=1, keepdims=True))


def _row_spec(width):
    return pl.BlockSpec((BLK, width), lambda i: (i, 0))


def _full_spec(shape):
    return pl.BlockSpec(shape, lambda i: tuple(0 for _ in shape))


def _tc_pre(xp, w1, w1s, acat):
    return pl.pallas_call(
        _tc_pre_body,
        grid=(NPAD // BLK,),
        in_specs=[_row_spec(128), _full_spec((128, 64)), _full_spec((128, 64)),
                  _full_spec((64, 16))],
        out_specs=[_row_spec(HBW), _row_spec(16)],
        out_shape=[jax.ShapeDtypeStruct((NPAD, HBW), jnp.bfloat16),
                   jax.ShapeDtypeStruct((NPAD, 16), jnp.float32)],
    )(xp, w1, w1s, acat)


def _tc_mid(u, b1r, w2p, w2s, a2, e16):
    return pl.pallas_call(
        _tc_mid_body,
        grid=(NPAD // BLK,),
        in_specs=[pl.BlockSpec((2, BLK, U1W), lambda i: (0, i, 0)),
                  _full_spec((1, 64)),
                  _full_spec((64, 48)), _full_spec((64, HBW)),
                  _full_spec((48, 16)), _full_spec((16, 64))],
        out_specs=[_row_spec(HBW), _row_spec(16)],
        out_shape=[jax.ShapeDtypeStruct((NPAD, HBW), jnp.bfloat16),
                   jax.ShapeDtypeStruct((NPAD, 16), jnp.float32)],
    )(u, b1r, w2p, w2s, a2, e16)


def _tc_post(u, b2r, e1):
    return pl.pallas_call(
        _tc_post_body,
        grid=(NPAD // BLK,),
        in_specs=[pl.BlockSpec((2, BLK, U2W), lambda i: (0, i, 0)),
                  _full_spec((1, 40)), _full_spec((8, 40))],
        out_specs=_row_spec(40),
        out_shape=jax.ShapeDtypeStruct((NPAD, 40), jnp.float32),
    )(u, b2r, e1)


def _vgather(v, idx):
    return v.at[idx].get(mode="promise_in_bounds")


def _edge1_compute(e, asds_v, asdd_v, hb_v, msg_v):
    iota = lax.iota(jnp.int32, 16)
    idx_hi = (iota & 7) + 8
    srow = asds_v[e]
    drow = asdd_v[e]
    ev = srow + _vgather(drow, idx_hi)
    ev = jnp.where(ev > 0, ev, 0.2 * ev)
    w = jnp.where(iota < 8, jnp.exp(ev), 0.0)
    for k in range(2):
        hb = hb_v[e, pl.ds(32 * k, 32)]
        a, b = plsc.unpack(hb, format=plsc.PackFormat.INTERLEAVED)
        wa = _vgather(w, jnp.right_shift(iota, 3) + 4 * k)
        wb = _vgather(w, jnp.right_shift(iota, 3) + 4 * k + 2)
        msg_v[e, pl.ds(32 * k, 16)] = a * wa
        msg_v[e, pl.ds(32 * k + 16, 16)] = b * wb
    msg_v[e, pl.ds(64, 16)] = w


def _edge2_compute(e, asds_v, asdd_v, hb_v, msg_v):
    iota = lax.iota(jnp.int32, 16)
    idx_hi = (iota & 7) + 8
    srow = asds_v[e]
    drow = asdd_v[e]
    ev = srow + _vgather(drow, idx_hi)
    ev = jnp.where(ev > 0, ev, 0.2 * ev)
    wb16 = _vgather(jnp.exp(ev), iota * 0)
    h0 = hb_v[e, pl.ds(0, 32)]
    a0, b0 = plsc.unpack(h0, format=plsc.PackFormat.INTERLEAVED)
    h1 = hb_v[e, pl.ds(32, 32)]
    a1, _b1 = plsc.unpack(h1, format=plsc.PackFormat.INTERLEAVED)
    msg_v[e, pl.ds(0, 16)] = a0 * wb16
    msg_v[e, pl.ds(16, 16)] = b0 * wb16
    msg_v[e, pl.ds(32, 16)] = a1 * wb16 + jnp.where(iota == 8, wb16, 0.0)


def _sc_edge_kernel(width_acc, per_edge_fn):
    def body(src_hbm, dst_hbm, asd_hbm, hb_hbm, out_hbm,
             src_v, dst_v, asds0_v, asds1_v, asdd0_v, asdd1_v,
             hb0_v, hb1_v, msg0_v, msg1_v,
             acc_sh, sg0, sg1, ss0, ss1):
        c = lax.axis_index("c")
        s = lax.axis_index("s")
        wid = c * NSUB + s

        pltpu.sync_copy(src_hbm.at[pl.ds(wid * CPT, CPT)], src_v)
        pltpu.sync_copy(dst_hbm.at[pl.ds(wid * CPT, CPT)], dst_v)

        asdss = (asds0_v, asds1_v)
        asdds = (asdd0_v, asdd1_v)
        hbs = (hb0_v, hb1_v)
        msgs = (msg0_v, msg1_v)
        sgs = (sg0, sg1)
        sss = (ss0, ss1)

        def issue(j, b):
            pltpu.async_copy(asd_hbm.at[src_v.at[j]], asdss[b], sgs[b])
            pltpu.async_copy(asd_hbm.at[dst_v.at[j]], asdds[b], sgs[b])
            pltpu.async_copy(hb_hbm.at[src_v.at[j]], hbs[b], sgs[b])

        def drain(j, b):
            pltpu.make_async_copy(asd_hbm.at[src_v.at[j]], asdss[b],
                                  sgs[b]).wait()
            pltpu.make_async_copy(asd_hbm.at[dst_v.at[j]], asdds[b],
                                  sgs[b]).wait()
            pltpu.make_async_copy(hb_hbm.at[src_v.at[j]], hbs[b],
                                  sgs[b]).wait()

        @plsc.parallel_loop(0, CHUNK, 1, unroll=4)
        def _zero_row(r):
            for k in range(width_acc // 16):
                msg0_v[r, pl.ds(16 * k, 16)] = jnp.zeros((16,), jnp.float32)
        for i in range(ROWS_PER_TILE // CHUNK):
            pltpu.sync_copy(msg0_v,
                            acc_sh.at[pl.ds(s * ROWS_PER_TILE + i * CHUNK,
                                            CHUNK)])
        plsc.subcore_barrier()

        issue(0, 0)

        def do_chunk(j, b):
            drain(j, b)

            @pl.when(j >= 2)
            def _():
                pltpu.make_async_copy(msgs[b], acc_sh.at[dst_v.at[j - 2]],
                                      sss[b]).wait()

            @plsc.parallel_loop(0, CHUNK, 1, unroll=4)
            def edge_body(e):
                per_edge_fn(e, asdss[b], asdds[b], hbs[b], msgs[b])

            pltpu.async_copy(msgs[b], acc_sh.at[dst_v.at[j]], sss[b],
                             add=True)

        def pair_body(k, _):
            j0 = 2 * k
            issue(j0 + 1, 1)
            do_chunk(j0, 0)

            @pl.when(k + 1 < CPT // 2)
            def _():
                issue(j0 + 2, 0)
            do_chunk(j0 + 1, 1)
            return 0

        lax.fori_loop(0, CPT // 2, pair_body, 0)
        pltpu.make_async_copy(msgs[0], acc_sh.at[dst_v.at[CPT - 2]],
                              sss[0]).wait()
        pltpu.make_async_copy(msgs[1], acc_sh.at[dst_v.at[CPT - 1]],
                              sss[1]).wait()
        plsc.subcore_barrier()

        pltpu.sync_copy(acc_sh.at[pl.ds(s * ROWS_PER_TILE, ROWS_PER_TILE)],
                        out_hbm.at[c, pl.ds(s * ROWS_PER_TILE, ROWS_PER_TILE)])

    mesh = plsc.VectorSubcoreMesh(core_axis_name="c", subcore_axis_name="s",
                                  num_cores=NCORE, num_subcores=NSUB)
    return pl.kernel(
        body,
        out_type=jax.ShapeDtypeStruct((NCORE, NPAD, width_acc), jnp.float32),
        mesh=mesh,
        compiler_params=pltpu.CompilerParams(use_tc_tiling_on_sc=False,
                                             needs_layout_passes=False),
        scratch_types=[
            pltpu.VMEM((CPT, CHUNK), jnp.int32),
            pltpu.VMEM((CPT, CHUNK), jnp.int32),
            pltpu.VMEM((CHUNK, 16), jnp.float32),
            pltpu.VMEM((CHUNK, 16), jnp.float32),
            pltpu.VMEM((CHUNK, 16), jnp.float32),
            pltpu.VMEM((CHUNK, 16), jnp.float32),
            pltpu.VMEM((CHUNK, HBW), jnp.bfloat16),
            pltpu.VMEM((CHUNK, HBW), jnp.bfloat16),
            pltpu.VMEM((CHUNK, width_acc), jnp.float32),
            pltpu.VMEM((CHUNK, width_acc), jnp.float32),
            pltpu.VMEM_SHARED((NPAD, width_acc), jnp.float32),
            pltpu.SemaphoreType.DMA,
            pltpu.SemaphoreType.DMA,
            pltpu.SemaphoreType.DMA,
            pltpu.SemaphoreType.DMA,
        ],
    )


def kernel(x, edge_index, W1, as1, ad1, b1, W2, as2, ad2, b2):
    f32 = jnp.float32

    loop = jnp.arange(NN, dtype=jnp.int32)
    pad = jnp.full((EPAD - ETOT,), NN, jnp.int32)
    src = jnp.concatenate([edge_index[0], loop, pad]).reshape(NTILE * CPT,
                                                              CHUNK)
    dst = jnp.concatenate([edge_index[1], loop, pad]).reshape(NTILE * CPT,
                                                              CHUNK)

    p = jnp.arange(HBW)
    perm = 32 * (p // 32) + (p % 32) // 2 + jnp.where(p % 2 == 1, 16, 0)

    j = jnp.arange(64)
    hd = j // 8
    acat = jnp.zeros((64, 16), f32)
    acat = acat.at[j, hd].set(as1.reshape(-1))
    acat = acat.at[j, hd + 8].set(ad1.reshape(-1))
    w1s = W1[:, perm]

    w2p = jnp.zeros((64, 48), f32).at[:, :40].set(W2)
    w2pad64 = jnp.zeros((64, HBW), f32).at[:, :40].set(W2)
    w2s = w2pad64[:, perm]
    a2 = jnp.zeros((48, 16), f32)
    a2 = a2.at[:40, 0:8].set(as2[0][:, None] * jnp.ones((40, 8), f32))
    a2 = a2.at[:40, 8:16].set(ad2[0][:, None] * jnp.ones((40, 8), f32))

    e16 = (jnp.arange(64)[None, :] // 8
           == jnp.arange(16)[:, None]).astype(f32)
    e1 = (jnp.arange(8)[:, None] == 0).astype(f32) * jnp.ones((8, 40), f32)

    hb1, asd1 = _tc_pre(x, W1, w1s, acat)

    u1 = _sc_edge_kernel(U1W, _edge1_compute)(src, dst, asd1, hb1)
    hb2, asd2 = _tc_mid(u1, b1.reshape(1, 64), w2p, w2s, a2, e16)

    u2 = _sc_edge_kernel(U2W, _edge2_compute)(src, dst, asd2, hb2)
    out = _tc_post(u2, b2.reshape(1, 40), e1)
    return out[:NN]


# speedup vs baseline: 1.1278x; 1.1278x over previous
import jax
import jax.numpy as jnp
from jax import lax
from jax.experimental import pallas as pl
from jax.experimental.pallas import tpu as pltpu
from jax.experimental.pallas import tpu_sc as plsc

NN = 10000
NPAD = 10240
EDGES = 320000
ETOT = EDGES + NN
NCORE = 2
NSUB = 16
NTILE = NCORE * NSUB
CHUNK = 128
CPT = 82
CPT0 = 94
CPT1 = 2 * CPT - CPT0
EPT = CPT * CHUNK
EPAD = (NSUB * (CPT0 + CPT1) + CPT0 - CPT1) * CHUNK
ROWS_PER_TILE = NPAD // NSUB

U1W = 80
U2W = 48
HBW = 64
BLK = 1024


def _tc_pre_body(x_ref, w1_ref, w1s_ref, acat_ref, hb_ref, asd_ref):
    h = jnp.dot(x_ref[...], w1_ref[...], preferred_element_type=jnp.float32)
    asd_ref[...] = jnp.dot(h, acat_ref[...], preferred_element_type=jnp.float32)
    hb_ref[...] = jnp.dot(x_ref[...], w1s_ref[...],
                          preferred_element_type=jnp.float32).astype(jnp.bfloat16)


def _tc_mid_body(u_ref, b1_ref, w2p_ref, w2s_ref, a2_ref, e16_ref,
                 hb_ref, asd2_ref):
    u = u_ref[0] + u_ref[1]
    den = jnp.dot(u[:, 64:80], e16_ref[...], preferred_element_type=jnp.float32)
    h1 = jnp.maximum(u[:, :64] / (den + 1e-16) + b1_ref[...], 0.0)
    h2 = jnp.dot(h1, w2p_ref[...], preferred_element_type=jnp.float32)
    asd2_ref[...] = jnp.dot(h2, a2_ref[...], preferred_element_type=jnp.float32)
    hb_ref[...] = jnp.dot(h1, w2s_ref[...],
                          preferred_element_type=jnp.float32).astype(jnp.bfloat16)


def _tc_post_body(u_ref, b2_ref, e1_ref, out_ref):
    u = u_ref[0] + u_ref[1]
    den = jnp.dot(u[:, 40:48], e1_ref[...], preferred_element_type=jnp.float32)
    logits = u[:, :40] / (den + 1e-16) + b2_ref[...]
    m = jnp.max(logits, axis=1, keepdims=True)
    p = logits - m
    out_ref[...] = p - jnp.log(jnp.sum(jnp.exp(p), axis=1, keepdims=True))


def _row_spec(width):
    return pl.BlockSpec((BLK, width), lambda i: (i, 0))


def _full_spec(shape):
    return pl.BlockSpec(shape, lambda i: tuple(0 for _ in shape))


def _tc_pre(xp, w1, w1s, acat):
    return pl.pallas_call(
        _tc_pre_body,
        grid=(NPAD // BLK,),
        in_specs=[_row_spec(128), _full_spec((128, 64)), _full_spec((128, 64)),
                  _full_spec((64, 16))],
        out_specs=[_row_spec(HBW), _row_spec(16)],
        out_shape=[jax.ShapeDtypeStruct((NPAD, HBW), jnp.bfloat16),
                   jax.ShapeDtypeStruct((NPAD, 16), jnp.float32)],
    )(xp, w1, w1s, acat)


def _tc_mid(u, b1r, w2p, w2s, a2, e16):
    return pl.pallas_call(
        _tc_mid_body,
        grid=(NPAD // BLK,),
        in_specs=[pl.BlockSpec((2, BLK, U1W), lambda i: (0, i, 0)),
                  _full_spec((1, 64)),
                  _full_spec((64, 48)), _full_spec((64, HBW)),
                  _full_spec((48, 16)), _full_spec((16, 64))],
        out_specs=[_row_spec(HBW), _row_spec(16)],
        out_shape=[jax.ShapeDtypeStruct((NPAD, HBW), jnp.bfloat16),
                   jax.ShapeDtypeStruct((NPAD, 16), jnp.float32)],
    )(u, b1r, w2p, w2s, a2, e16)


def _tc_post(u, b2r, e1):
    return pl.pallas_call(
        _tc_post_body,
        grid=(NPAD // BLK,),
        in_specs=[pl.BlockSpec((2, BLK, U2W), lambda i: (0, i, 0)),
                  _full_spec((1, 40)), _full_spec((8, 40))],
        out_specs=_row_spec(40),
        out_shape=jax.ShapeDtypeStruct((NPAD, 40), jnp.float32),
    )(u, b2r, e1)


def _vgather(v, idx):
    return v.at[idx].get(mode="promise_in_bounds")


def _edge1_compute(e, asds_v, asdd_v, hb_v, msg_v):
    iota = lax.iota(jnp.int32, 16)
    idx_hi = (iota & 7) + 8
    srow = asds_v[e]
    drow = asdd_v[e]
    ev = srow + _vgather(drow, idx_hi)
    ev = jnp.where(ev > 0, ev, 0.2 * ev)
    w = jnp.where(iota < 8, jnp.exp(ev), 0.0)
    for k in range(2):
        hb = hb_v[e, pl.ds(32 * k, 32)]
        a, b = plsc.unpack(hb, format=plsc.PackFormat.INTERLEAVED)
        wa = _vgather(w, jnp.right_shift(iota, 3) + 4 * k)
        wb = _vgather(w, jnp.right_shift(iota, 3) + 4 * k + 2)
        msg_v[e, pl.ds(32 * k, 16)] = a * wa
        msg_v[e, pl.ds(32 * k + 16, 16)] = b * wb
    msg_v[e, pl.ds(64, 16)] = w


def _edge2_compute(e, asds_v, asdd_v, hb_v, msg_v):
    iota = lax.iota(jnp.int32, 16)
    idx_hi = (iota & 7) + 8
    srow = asds_v[e]
    drow = asdd_v[e]
    ev = srow + _vgather(drow, idx_hi)
    ev = jnp.where(ev > 0, ev, 0.2 * ev)
    wb16 = _vgather(jnp.exp(ev), iota * 0)
    h0 = hb_v[e, pl.ds(0, 32)]
    a0, b0 = plsc.unpack(h0, format=plsc.PackFormat.INTERLEAVED)
    h1 = hb_v[e, pl.ds(32, 32)]
    a1, _b1 = plsc.unpack(h1, format=plsc.PackFormat.INTERLEAVED)
    msg_v[e, pl.ds(0, 16)] = a0 * wb16
    msg_v[e, pl.ds(16, 16)] = b0 * wb16
    msg_v[e, pl.ds(32, 16)] = a1 * wb16 + jnp.where(iota == 8, wb16, 0.0)


def _sc_edge_kernel(width_acc, per_edge_fn):
    def body(src_hbm, dst_hbm, asd_hbm, hb_hbm, out_hbm,
             src_v, dst_v, asds0_v, asds1_v, asdd0_v, asdd1_v,
             hb0_v, hb1_v, msg0_v, msg1_v,
             acc_sh, sg0, sg1, ss0, ss1):
        c = lax.axis_index("c")
        s = lax.axis_index("s")
        row_base = jnp.where(c == 0, s * CPT0, NSUB * CPT0 + s * CPT1)
        cpt_mine = jnp.where(c == 0, CPT0, CPT1)

        pltpu.sync_copy(src_hbm.at[pl.ds(row_base, CPT0)], src_v)
        pltpu.sync_copy(dst_hbm.at[pl.ds(row_base, CPT0)], dst_v)

        asdss = (asds0_v, asds1_v)
        asdds = (asdd0_v, asdd1_v)
        hbs = (hb0_v, hb1_v)
        msgs = (msg0_v, msg1_v)
        sgs = (sg0, sg1)
        sss = (ss0, ss1)

        def issue(j, b):
            pltpu.async_copy(asd_hbm.at[src_v.at[j]], asdss[b], sgs[b])
            pltpu.async_copy(asd_hbm.at[dst_v.at[j]], asdds[b], sgs[b])
            pltpu.async_copy(hb_hbm.at[src_v.at[j]], hbs[b], sgs[b])

        def drain(j, b):
            pltpu.make_async_copy(asd_hbm.at[src_v.at[j]], asdss[b],
                                  sgs[b]).wait()
            pltpu.make_async_copy(asd_hbm.at[dst_v.at[j]], asdds[b],
                                  sgs[b]).wait()
            pltpu.make_async_copy(hb_hbm.at[src_v.at[j]], hbs[b],
                                  sgs[b]).wait()

        @plsc.parallel_loop(0, CHUNK, 1, unroll=4)
        def _zero_row(r):
            for k in range(width_acc // 16):
                msg0_v[r, pl.ds(16 * k, 16)] = jnp.zeros((16,), jnp.float32)
        for i in range(ROWS_PER_TILE // CHUNK):
            pltpu.sync_copy(msg0_v,
                            acc_sh.at[pl.ds(s * ROWS_PER_TILE + i * CHUNK,
                                            CHUNK)])
        plsc.subcore_barrier()

        issue(0, 0)

        def do_chunk(j, b):
            drain(j, b)

            @pl.when(j >= 2)
            def _():
                pltpu.make_async_copy(msgs[b], acc_sh.at[dst_v.at[j - 2]],
                                      sss[b]).wait()

            @plsc.parallel_loop(0, CHUNK, 1, unroll=4)
            def edge_body(e):
                per_edge_fn(e, asdss[b], asdds[b], hbs[b], msgs[b])

            pltpu.async_copy(msgs[b], acc_sh.at[dst_v.at[j]], sss[b],
                             add=True)

        def pair_body(k, _):
            j0 = 2 * k
            issue(j0 + 1, 1)
            do_chunk(j0, 0)

            @pl.when(k + 1 < cpt_mine // 2)
            def _():
                issue(j0 + 2, 0)
            do_chunk(j0 + 1, 1)
            return 0

        lax.fori_loop(0, cpt_mine // 2, pair_body, 0)
        pltpu.make_async_copy(msgs[0], acc_sh.at[dst_v.at[cpt_mine - 2]],
                              sss[0]).wait()
        pltpu.make_async_copy(msgs[1], acc_sh.at[dst_v.at[cpt_mine - 1]],
                              sss[1]).wait()
        plsc.subcore_barrier()

        pltpu.sync_copy(acc_sh.at[pl.ds(s * ROWS_PER_TILE, ROWS_PER_TILE)],
                        out_hbm.at[c, pl.ds(s * ROWS_PER_TILE, ROWS_PER_TILE)])

    mesh = plsc.VectorSubcoreMesh(core_axis_name="c", subcore_axis_name="s",
                                  num_cores=NCORE, num_subcores=NSUB)
    return pl.kernel(
        body,
        out_type=jax.ShapeDtypeStruct((NCORE, NPAD, width_acc), jnp.float32),
        mesh=mesh,
        compiler_params=pltpu.CompilerParams(use_tc_tiling_on_sc=False,
                                             needs_layout_passes=False),
        scratch_types=[
            pltpu.VMEM((CPT0, CHUNK), jnp.int32),
            pltpu.VMEM((CPT0, CHUNK), jnp.int32),
            pltpu.VMEM((CHUNK, 16), jnp.float32),
            pltpu.VMEM((CHUNK, 16), jnp.float32),
            pltpu.VMEM((CHUNK, 16), jnp.float32),
            pltpu.VMEM((CHUNK, 16), jnp.float32),
            pltpu.VMEM((CHUNK, HBW), jnp.bfloat16),
            pltpu.VMEM((CHUNK, HBW), jnp.bfloat16),
            pltpu.VMEM((CHUNK, width_acc), jnp.float32),
            pltpu.VMEM((CHUNK, width_acc), jnp.float32),
            pltpu.VMEM_SHARED((NPAD, width_acc), jnp.float32),
            pltpu.SemaphoreType.DMA,
            pltpu.SemaphoreType.DMA,
            pltpu.SemaphoreType.DMA,
            pltpu.SemaphoreType.DMA,
        ],
    )


def kernel(x, edge_index, W1, as1, ad1, b1, W2, as2, ad2, b2):
    f32 = jnp.float32

    loop = jnp.arange(NN, dtype=jnp.int32)
    pad = jnp.full((EPAD - ETOT,), NN, jnp.int32)
    src = jnp.concatenate([edge_index[0], loop, pad]).reshape(-1, CHUNK)
    dst = jnp.concatenate([edge_index[1], loop, pad]).reshape(-1, CHUNK)

    p = jnp.arange(HBW)
    perm = 32 * (p // 32) + (p % 32) // 2 + jnp.where(p % 2 == 1, 16, 0)

    j = jnp.arange(64)
    hd = j // 8
    acat = jnp.zeros((64, 16), f32)
    acat = acat.at[j, hd].set(as1.reshape(-1))
    acat = acat.at[j, hd + 8].set(ad1.reshape(-1))
    w1s = W1[:, perm]

    w2p = jnp.zeros((64, 48), f32).at[:, :40].set(W2)
    w2pad64 = jnp.zeros((64, HBW), f32).at[:, :40].set(W2)
    w2s = w2pad64[:, perm]
    a2 = jnp.zeros((48, 16), f32)
    a2 = a2.at[:40, 0:8].set(as2[0][:, None] * jnp.ones((40, 8), f32))
    a2 = a2.at[:40, 8:16].set(ad2[0][:, None] * jnp.ones((40, 8), f32))

    e16 = (jnp.arange(64)[None, :] // 8
           == jnp.arange(16)[:, None]).astype(f32)
    e1 = (jnp.arange(8)[:, None] == 0).astype(f32) * jnp.ones((8, 40), f32)

    hb1, asd1 = _tc_pre(x, W1, w1s, acat)

    u1 = _sc_edge_kernel(U1W, _edge1_compute)(src, dst, asd1, hb1)
    hb2, asd2 = _tc_mid(u1, b1.reshape(1, 64), w2p, w2s, a2, e16)

    u2 = _sc_edge_kernel(U2W, _edge2_compute)(src, dst, asd2, hb2)
    out = _tc_post(u2, b2.reshape(1, 40), e1)
    return out[:NN]


# speedup vs baseline: 1.1341x; 1.0056x over previous
import jax
import jax.numpy as jnp
from jax import lax
from jax.experimental import pallas as pl
from jax.experimental.pallas import tpu as pltpu
from jax.experimental.pallas import tpu_sc as plsc

NN = 10000
NPAD = 10240
EDGES = 320000
ETOT = EDGES + NN
NCORE = 2
NSUB = 16
NTILE = NCORE * NSUB
CHUNK = 128
CPT = 82
CPT0 = 100
CPT1 = 2 * CPT - CPT0
EPT = CPT * CHUNK
EPAD = (NSUB * (CPT0 + CPT1) + CPT0 - CPT1) * CHUNK
ROWS_PER_TILE = NPAD // NSUB

U1W = 80
U2W = 48
HBW = 64
BLK = 1024


def _tc_pre_body(x_ref, w1_ref, w1s_ref, acat_ref, hb_ref, asd_ref):
    h = jnp.dot(x_ref[...], w1_ref[...], preferred_element_type=jnp.float32)
    asd_ref[...] = jnp.dot(h, acat_ref[...], preferred_element_type=jnp.float32)
    hb_ref[...] = jnp.dot(x_ref[...], w1s_ref[...],
                          preferred_element_type=jnp.float32).astype(jnp.bfloat16)


def _tc_mid_body(u_ref, b1_ref, w2p_ref, w2s_ref, a2_ref, e16_ref,
                 hb_ref, asd2_ref):
    u = u_ref[0] + u_ref[1]
    den = jnp.dot(u[:, 64:80], e16_ref[...], preferred_element_type=jnp.float32)
    h1 = jnp.maximum(u[:, :64] / (den + 1e-16) + b1_ref[...], 0.0)
    h2 = jnp.dot(h1, w2p_ref[...], preferred_element_type=jnp.float32)
    asd2_ref[...] = jnp.dot(h2, a2_ref[...], preferred_element_type=jnp.float32)
    hb_ref[...] = jnp.dot(h1, w2s_ref[...],
                          preferred_element_type=jnp.float32).astype(jnp.bfloat16)


def _tc_post_body(u_ref, b2_ref, e1_ref, out_ref):
    u = u_ref[0] + u_ref[1]
    den = jnp.dot(u[:, 40:48], e1_ref[...], preferred_element_type=jnp.float32)
    logits = u[:, :40] / (den + 1e-16) + b2_ref[...]
    m = jnp.max(logits, axis=1, keepdims=True)
    p = logits - m
    out_ref[...] = p - jnp.log(jnp.sum(jnp.exp(p), axis=1, keepdims=True))


def _row_spec(width):
    return pl.BlockSpec((BLK, width), lambda i: (i, 0))


def _full_spec(shape):
    return pl.BlockSpec(shape, lambda i: tuple(0 for _ in shape))


def _tc_pre(xp, w1, w1s, acat):
    return pl.pallas_call(
        _tc_pre_body,
        grid=(NPAD // BLK,),
        in_specs=[_row_spec(128), _full_spec((128, 64)), _full_spec((128, 64)),
                  _full_spec((64, 16))],
        out_specs=[_row_spec(HBW), _row_spec(16)],
        out_shape=[jax.ShapeDtypeStruct((NPAD, HBW), jnp.bfloat16),
                   jax.ShapeDtypeStruct((NPAD, 16), jnp.float32)],
    )(xp, w1, w1s, acat)


def _tc_mid(u, b1r, w2p, w2s, a2, e16):
    return pl.pallas_call(
        _tc_mid_body,
        grid=(NPAD // BLK,),
        in_specs=[pl.BlockSpec((2, BLK, U1W), lambda i: (0, i, 0)),
                  _full_spec((1, 64)),
                  _full_spec((64, 48)), _full_spec((64, HBW)),
                  _full_spec((48, 16)), _full_spec((16, 64))],
        out_specs=[_row_spec(HBW), _row_spec(16)],
        out_shape=[jax.ShapeDtypeStruct((NPAD, HBW), jnp.bfloat16),
                   jax.ShapeDtypeStruct((NPAD, 16), jnp.float32)],
    )(u, b1r, w2p, w2s, a2, e16)


def _tc_post(u, b2r, e1):
    return pl.pallas_call(
        _tc_post_body,
        grid=(NPAD // BLK,),
        in_specs=[pl.BlockSpec((2, BLK, U2W), lambda i: (0, i, 0)),
                  _full_spec((1, 40)), _full_spec((8, 40))],
        out_specs=_row_spec(40),
        out_shape=jax.ShapeDtypeStruct((NPAD, 40), jnp.float32),
    )(u, b2r, e1)


def _vgather(v, idx):
    return v.at[idx].get(mode="promise_in_bounds")


def _edge1_compute(e, asds_v, asdd_v, hb_v, msg_v):
    iota = lax.iota(jnp.int32, 16)
    idx_hi = (iota & 7) + 8
    srow = asds_v[e]
    drow = asdd_v[e]
    ev = srow + _vgather(drow, idx_hi)
    ev = jnp.where(ev > 0, ev, 0.2 * ev)
    w = jnp.where(iota < 8, jnp.exp(ev), 0.0)
    for k in range(2):
        hb = hb_v[e, pl.ds(32 * k, 32)]
        a, b = plsc.unpack(hb, format=plsc.PackFormat.INTERLEAVED)
        wa = _vgather(w, jnp.right_shift(iota, 3) + 4 * k)
        wb = _vgather(w, jnp.right_shift(iota, 3) + 4 * k + 2)
        msg_v[e, pl.ds(32 * k, 16)] = a * wa
        msg_v[e, pl.ds(32 * k + 16, 16)] = b * wb
    msg_v[e, pl.ds(64, 16)] = w


def _edge2_compute(e, asds_v, asdd_v, hb_v, msg_v):
    iota = lax.iota(jnp.int32, 16)
    idx_hi = (iota & 7) + 8
    srow = asds_v[e]
    drow = asdd_v[e]
    ev = srow + _vgather(drow, idx_hi)
    ev = jnp.where(ev > 0, ev, 0.2 * ev)
    wb16 = _vgather(jnp.exp(ev), iota * 0)
    h0 = hb_v[e, pl.ds(0, 32)]
    a0, b0 = plsc.unpack(h0, format=plsc.PackFormat.INTERLEAVED)
    h1 = hb_v[e, pl.ds(32, 32)]
    a1, _b1 = plsc.unpack(h1, format=plsc.PackFormat.INTERLEAVED)
    msg_v[e, pl.ds(0, 16)] = a0 * wb16
    msg_v[e, pl.ds(16, 16)] = b0 * wb16
    msg_v[e, pl.ds(32, 16)] = a1 * wb16 + jnp.where(iota == 8, wb16, 0.0)


def _sc_edge_kernel(width_acc, per_edge_fn):
    def body(src_hbm, dst_hbm, asd_hbm, hb_hbm, out_hbm,
             src_v, dst_v, asds0_v, asds1_v, asdd0_v, asdd1_v,
             hb0_v, hb1_v, msg0_v, msg1_v,
             acc_sh, sg0, sg1, ss0, ss1):
        c = lax.axis_index("c")
        s = lax.axis_index("s")
        row_base = jnp.where(c == 0, s * CPT0, NSUB * CPT0 + s * CPT1)
        cpt_mine = jnp.where(c == 0, CPT0, CPT1)

        pltpu.sync_copy(src_hbm.at[pl.ds(row_base, CPT0)], src_v)
        pltpu.sync_copy(dst_hbm.at[pl.ds(row_base, CPT0)], dst_v)

        asdss = (asds0_v, asds1_v)
        asdds = (asdd0_v, asdd1_v)
        hbs = (hb0_v, hb1_v)
        msgs = (msg0_v, msg1_v)
        sgs = (sg0, sg1)
        sss = (ss0, ss1)

        def issue(j, b):
            pltpu.async_copy(asd_hbm.at[src_v.at[j]], asdss[b], sgs[b])
            pltpu.async_copy(asd_hbm.at[dst_v.at[j]], asdds[b], sgs[b])
            pltpu.async_copy(hb_hbm.at[src_v.at[j]], hbs[b], sgs[b])

        def drain(j, b):
            pltpu.make_async_copy(asd_hbm.at[src_v.at[j]], asdss[b],
                                  sgs[b]).wait()
            pltpu.make_async_copy(asd_hbm.at[dst_v.at[j]], asdds[b],
                                  sgs[b]).wait()
            pltpu.make_async_copy(hb_hbm.at[src_v.at[j]], hbs[b],
                                  sgs[b]).wait()

        @plsc.parallel_loop(0, CHUNK, 1, unroll=4)
        def _zero_row(r):
            for k in range(width_acc // 16):
                msg0_v[r, pl.ds(16 * k, 16)] = jnp.zeros((16,), jnp.float32)
        for i in range(ROWS_PER_TILE // CHUNK):
            pltpu.sync_copy(msg0_v,
                            acc_sh.at[pl.ds(s * ROWS_PER_TILE + i * CHUNK,
                                            CHUNK)])
        plsc.subcore_barrier()

        issue(0, 0)

        def do_chunk(j, b):
            drain(j, b)

            @pl.when(j >= 2)
            def _():
                pltpu.make_async_copy(msgs[b], acc_sh.at[dst_v.at[j - 2]],
                                      sss[b]).wait()

            @plsc.parallel_loop(0, CHUNK, 1, unroll=4)
            def edge_body(e):
                per_edge_fn(e, asdss[b], asdds[b], hbs[b], msgs[b])

            pltpu.async_copy(msgs[b], acc_sh.at[dst_v.at[j]], sss[b],
                             add=True)

        def pair_body(k, _):
            j0 = 2 * k
            issue(j0 + 1, 1)
            do_chunk(j0, 0)

            @pl.when(k + 1 < cpt_mine // 2)
            def _():
                issue(j0 + 2, 0)
            do_chunk(j0 + 1, 1)
            return 0

        lax.fori_loop(0, cpt_mine // 2, pair_body, 0)
        pltpu.make_async_copy(msgs[0], acc_sh.at[dst_v.at[cpt_mine - 2]],
                              sss[0]).wait()
        pltpu.make_async_copy(msgs[1], acc_sh.at[dst_v.at[cpt_mine - 1]],
                              sss[1]).wait()
        plsc.subcore_barrier()

        pltpu.sync_copy(acc_sh.at[pl.ds(s * ROWS_PER_TILE, ROWS_PER_TILE)],
                        out_hbm.at[c, pl.ds(s * ROWS_PER_TILE, ROWS_PER_TILE)])

    mesh = plsc.VectorSubcoreMesh(core_axis_name="c", subcore_axis_name="s",
                                  num_cores=NCORE, num_subcores=NSUB)
    return pl.kernel(
        body,
        out_type=jax.ShapeDtypeStruct((NCORE, NPAD, width_acc), jnp.float32),
        mesh=mesh,
        compiler_params=pltpu.CompilerParams(use_tc_tiling_on_sc=False,
                                             needs_layout_passes=False),
        scratch_types=[
            pltpu.VMEM((CPT0, CHUNK), jnp.int32),
            pltpu.VMEM((CPT0, CHUNK), jnp.int32),
            pltpu.VMEM((CHUNK, 16), jnp.float32),
            pltpu.VMEM((CHUNK, 16), jnp.float32),
            pltpu.VMEM((CHUNK, 16), jnp.float32),
            pltpu.VMEM((CHUNK, 16), jnp.float32),
            pltpu.VMEM((CHUNK, HBW), jnp.bfloat16),
            pltpu.VMEM((CHUNK, HBW), jnp.bfloat16),
            pltpu.VMEM((CHUNK, width_acc), jnp.float32),
            pltpu.VMEM((CHUNK, width_acc), jnp.float32),
            pltpu.VMEM_SHARED((NPAD, width_acc), jnp.float32),
            pltpu.SemaphoreType.DMA,
            pltpu.SemaphoreType.DMA,
            pltpu.SemaphoreType.DMA,
            pltpu.SemaphoreType.DMA,
        ],
    )


def kernel(x, edge_index, W1, as1, ad1, b1, W2, as2, ad2, b2):
    f32 = jnp.float32

    loop = jnp.arange(NN, dtype=jnp.int32)
    pad = jnp.full((EPAD - ETOT,), NN, jnp.int32)
    src = jnp.concatenate([edge_index[0], loop, pad]).reshape(-1, CHUNK)
    dst = jnp.concatenate([edge_index[1], loop, pad]).reshape(-1, CHUNK)

    p = jnp.arange(HBW)
    perm = 32 * (p // 32) + (p % 32) // 2 + jnp.where(p % 2 == 1, 16, 0)

    j = jnp.arange(64)
    hd = j // 8
    acat = jnp.zeros((64, 16), f32)
    acat = acat.at[j, hd].set(as1.reshape(-1))
    acat = acat.at[j, hd + 8].set(ad1.reshape(-1))
    w1s = W1[:, perm]

    w2p = jnp.zeros((64, 48), f32).at[:, :40].set(W2)
    w2pad64 = jnp.zeros((64, HBW), f32).at[:, :40].set(W2)
    w2s = w2pad64[:, perm]
    a2 = jnp.zeros((48, 16), f32)
    a2 = a2.at[:40, 0:8].set(as2[0][:, None] * jnp.ones((40, 8), f32))
    a2 = a2.at[:40, 8:16].set(ad2[0][:, None] * jnp.ones((40, 8), f32))

    e16 = (jnp.arange(64)[None, :] // 8
           == jnp.arange(16)[:, None]).astype(f32)
    e1 = (jnp.arange(8)[:, None] == 0).astype(f32) * jnp.ones((8, 40), f32)

    hb1, asd1 = _tc_pre(x, W1, w1s, acat)

    u1 = _sc_edge_kernel(U1W, _edge1_compute)(src, dst, asd1, hb1)
    hb2, asd2 = _tc_mid(u1, b1.reshape(1, 64), w2p, w2s, a2, e16)

    u2 = _sc_edge_kernel(U2W, _edge2_compute)(src, dst, asd2, hb2)
    out = _tc_post(u2, b2.reshape(1, 40), e1)
    return out[:NN]
